# baseline (speedup 1.0000x reference)
import functools

import jax
import jax.numpy as jnp
from jax import lax
from jax.experimental import pallas as pl
from jax.experimental.pallas import tpu as pltpu
from jax.experimental.pallas import tpu_sc as plsc

_NC, _NS, _L = 2, 16, 16
_N = 10000
_NP = 10240
_D = 128
_CH = 64
_NCH = 160
_EG_PAD = _NS * _NCH * _CH
_EPT = _EG_PAD // _NS


def _gkt_body(nf, src_h, tgt_h, w_h, msg1_o, diff_o,
              acc, g0, g1, s0, s1, sv0, sv1, tv0, tv1, tv2, tv3,
              wv0, wv1, wv2, wv3,
              gsem0, gsem1, ssem0, ssem1, asem0, asem1):
    c = lax.axis_index("c")
    s = lax.axis_index("s")
    zero16 = jnp.zeros((_L,), jnp.float32)
    gb = (g0, g1)
    sb = (s0, s1)
    sbuf = (sv0, sv1)
    tbuf = (tv0, tv1, tv2, tv3)
    wbuf = (wv0, wv1, wv2, wv3)
    gsem = (gsem0, gsem1)
    ssem = (ssem0, ssem1)
    asem = (asem0, asem1)

    def memset_row(r, carry):
        for j in range(_D // _L):
            s0[r, pl.ds(j * _L, _L)] = zero16
        return carry

    lax.fori_loop(0, _CH, memset_row, 0)
    rpw = _NP // _NS
    w0 = s * rpw
    for i in range(rpw // _CH):
        pltpu.sync_copy(s0, acc.at[pl.ds(w0 + i * _CH, _CH)])
    plsc.subcore_barrier()

    base = s * _EPT

    def issue_smalls(k, b, q):
        off = base + k * _CH
        pltpu.async_copy(src_h.at[pl.ds(off, _CH)], sbuf[b], asem[b])
        pltpu.async_copy(tgt_h.at[pl.ds(off, _CH)], tbuf[q], asem[b])
        pltpu.async_copy(w_h.at[c, pl.ds(off, _CH)], wbuf[q], asem[b])

    def wait_smalls(k, b, q):
        off = base + k * _CH
        pltpu.make_async_copy(src_h.at[pl.ds(off, _CH)], sbuf[b], asem[b]).wait()
        pltpu.make_async_copy(tgt_h.at[pl.ds(off, _CH)], tbuf[q], asem[b]).wait()
        pltpu.make_async_copy(w_h.at[c, pl.ds(off, _CH)], wbuf[q], asem[b]).wait()

    def issue_gather(b):
        pltpu.async_copy(nf.at[sbuf[b]], gb[b], gsem[b])

    def wait_gather(b):
        pltpu.make_async_copy(nf.at[sbuf[b]], gb[b], gsem[b]).wait()

    def issue_scat(b, q):
        pltpu.async_copy(sb[b], acc.at[tbuf[q]], ssem[b], add=True)

    def wait_scat(b, q):
        pltpu.make_async_copy(sb[b], acc.at[tbuf[q]], ssem[b]).wait()

    def scale(b, q):
        gbuf = gb[b]
        obuf = sb[b]
        wv = wbuf[q]

        def grp(g, carry):
            wvec = wv[pl.ds(g * _L, _L)]
            for rl in range(_L):
                r = g * _L + rl
                sp = jnp.full((_L,), wvec[rl], dtype=jnp.float32)
                for j in range(_D // _L):
                    sl = pl.ds(j * _L, _L)
                    obuf[r, sl] = gbuf[r, sl] * sp
            return carry

        lax.fori_loop(0, _CH // _L, grp, 0)

    def step(k, q, first=False, guard=True):
        b = q % 2
        wait_gather(b)

        if not first:
            wait_scat(b, q)

        if guard:
            @pl.when(k + 2 < _NCH)
            def _():
                issue_smalls(k + 2, b, (q + 2) % 4)
        else:
            issue_smalls(k + 2, b, (q + 2) % 4)

        if guard:
            @pl.when(k + 1 < _NCH)
            def _():
                wait_smalls(k + 1, 1 - b, (q + 1) % 4)
                issue_gather(1 - b)
        else:
            wait_smalls(k + 1, 1 - b, (q + 1) % 4)
            issue_gather(1 - b)

        scale(b, q)
        issue_scat(b, q)

    issue_smalls(0, 0, 0)
    issue_smalls(1, 1, 1)
    wait_smalls(0, 0, 0)
    issue_gather(0)

    step(0, 0, first=True, guard=False)
    step(1, 1, first=True, guard=False)
    step(2, 2, guard=False)
    step(3, 3, guard=False)

    def pipe(j, carry):
        k = j * 4
        step(k, 0)
        step(k + 1, 1)
        step(k + 2, 2)
        step(k + 3, 3)
        return carry

    lax.fori_loop(1, _NCH // 4, pipe, 0)
    wait_scat(0, 2)
    wait_scat(1, 3)
    plsc.subcore_barrier()

    @pl.when(c == 0)
    def _():
        pltpu.sync_copy(acc.at[pl.ds(w0, rpw)], msg1_o.at[pl.ds(w0, rpw)])

    @pl.when(c == 1)
    def _():
        pltpu.sync_copy(acc.at[pl.ds(w0, rpw)], diff_o.at[pl.ds(w0, rpw)])


@jax.jit
def _gkt_pass(nf, src, tgt, w2):
    run = pl.kernel(
        _gkt_body,
        out_type=[
            jax.ShapeDtypeStruct((_NP, _D), jnp.float32),
            jax.ShapeDtypeStruct((_NP, _D), jnp.float32),
        ],
        mesh=plsc.VectorSubcoreMesh(core_axis_name="c", subcore_axis_name="s"),
        scratch_types=[
            pltpu.VMEM_SHARED((_NP, _D), jnp.float32),
            pltpu.VMEM((_CH, _D), jnp.float32),
            pltpu.VMEM((_CH, _D), jnp.float32),
            pltpu.VMEM((_CH, _D), jnp.float32),
            pltpu.VMEM((_CH, _D), jnp.float32),
            pltpu.VMEM((_CH,), jnp.int32),
            pltpu.VMEM((_CH,), jnp.int32),
            pltpu.VMEM((_CH,), jnp.int32),
            pltpu.VMEM((_CH,), jnp.int32),
            pltpu.VMEM((_CH,), jnp.int32),
            pltpu.VMEM((_CH,), jnp.int32),
            pltpu.VMEM((_CH,), jnp.float32),
            pltpu.VMEM((_CH,), jnp.float32),
            pltpu.VMEM((_CH,), jnp.float32),
            pltpu.VMEM((_CH,), jnp.float32),
            pltpu.SemaphoreType.DMA,
            pltpu.SemaphoreType.DMA,
            pltpu.SemaphoreType.DMA,
            pltpu.SemaphoreType.DMA,
            pltpu.SemaphoreType.DMA,
            pltpu.SemaphoreType.DMA,
        ],
    )
    return run(nf, src, tgt, w2)


_EC_PAD = 20480
_RPW = _NP // (_NC * _NS)
_ACC_R = _RPW + 8
_GCH = 64


def _cfg_body(msg1_h, hid_h, cs_h, ct_h, st_h, h2_o,
              acc, stv, eidx, etgt, g1, g2, sem):
    c = lax.axis_index("c")
    s = lax.axis_index("s")
    wid = s * _NC + c
    lo = wid * _RPW
    ninf = jnp.full((_L,), -jnp.inf, dtype=jnp.float32)
    zerof = jnp.zeros((_L,), dtype=jnp.float32)
    lov = jnp.full((_L,), lo, dtype=jnp.int32)
    onev = jnp.full((_L,), 1, dtype=jnp.int32)
    zerov = jnp.full((_L,), 0, dtype=jnp.int32)
    rpwv = jnp.full((_L,), _RPW, dtype=jnp.int32)

    def init_row(r, carry):
        for j in range(2 * _D // _L):
            acc[r, pl.ds(j * _L, _L)] = ninf
        return carry

    lax.fori_loop(0, _ACC_R, init_row, 0)

    pltpu.sync_copy(st_h, stv)
    sv_ = stv[pl.ds(wid, _L)]
    s_lo = sv_[0]
    s_hi = sv_[1]
    a_lo = (s_lo // 8) * 8
    ng = (s_hi - a_lo + _GCH - 1) // _GCH

    def grp(g, carry):
        off = a_lo + g * _GCH
        pltpu.sync_copy(cs_h.at[pl.ds(off, _GCH)], eidx)
        pltpu.sync_copy(ct_h.at[pl.ds(off, _GCH)], etgt)
        pltpu.async_copy(msg1_h.at[eidx], g1, sem).wait()
        pltpu.async_copy(hid_h.at[eidx], g2, sem).wait()
        for vq in range(_GCH // _L):
            tv = etgt[pl.ds(vq * _L, _L)]
            d = tv - lov
            ge0 = jnp.minimum(jnp.maximum(d + onev, zerov), onev)
            ltr = jnp.minimum(jnp.maximum(rpwv - d, zerov), onev)
            mi = ge0 * ltr
            tlv = mi * d + (onev - mi) * rpwv
            for rl in range(_L):
                r = vq * _L + rl
                tl = tlv[rl]
                for j in range(_D // _L):
                    sl = pl.ds(j * _L, _L)
                    acc[tl, sl] = jnp.maximum(acc[tl, sl], g1[r, sl])
                for j in range(_D // _L):
                    sl = pl.ds(j * _L, _L)
                    sl2 = pl.ds(_D + j * _L, _L)
                    acc[tl, sl2] = jnp.maximum(acc[tl, sl2], g2[r, sl])
        return carry

    lax.fori_loop(0, ng, grp, 0)

    def fin_row(r, carry):
        for j in range(2 * _D // _L):
            sl = pl.ds(j * _L, _L)
            v = acc[r, sl]
            acc[r, sl] = jnp.where(v > ninf, v, zerof)
        return carry

    lax.fori_loop(0, _RPW, fin_row, 0)
    pltpu.sync_copy(acc.at[pl.ds(0, _RPW)], h2_o.at[pl.ds(lo, _RPW)])


@jax.jit
def _cfg_pass(msg1, hid, cs, ct, starts):
    run = pl.kernel(
        _cfg_body,
        out_type=jax.ShapeDtypeStruct((_NP, 2 * _D), jnp.float32),
        mesh=plsc.VectorSubcoreMesh(core_axis_name="c", subcore_axis_name="s"),
        scratch_types=[
            pltpu.VMEM((_ACC_R, 2 * _D), jnp.float32),
            pltpu.VMEM((48,), jnp.int32),
            pltpu.VMEM((_GCH,), jnp.int32),
            pltpu.VMEM((_GCH,), jnp.int32),
            pltpu.VMEM((_GCH, _D), jnp.float32),
            pltpu.VMEM((_GCH, _D), jnp.float32),
            pltpu.SemaphoreType.DMA,
        ],
    )
    return run(msg1, hid, cs, ct, starts)


def _proj_body(h2_ref, diff_ref, w2_ref, wd_ref, b_ref, o_ref):
    acc = jnp.dot(h2_ref[...], w2_ref[...], preferred_element_type=jnp.float32)
    acc = acc + jnp.dot(diff_ref[...], wd_ref[...], preferred_element_type=jnp.float32)
    o_ref[...] = jnp.maximum(acc + b_ref[...], 0.0)


def _fused_projection(h2, diff, W, b):
    NP, twoD = h2.shape
    D = diff.shape[1]
    OUT = W.shape[1]
    BN = 512
    return pl.pallas_call(
        _proj_body,
        grid=(NP // BN,),
        in_specs=[
            pl.BlockSpec((BN, twoD), lambda i: (i, 0)),
            pl.BlockSpec((BN, D), lambda i: (i, 0)),
            pl.BlockSpec((twoD, OUT), lambda i: (0, 0)),
            pl.BlockSpec((D, OUT), lambda i: (0, 0)),
            pl.BlockSpec((1, OUT), lambda i: (0, 0)),
        ],
        out_specs=pl.BlockSpec((BN, OUT), lambda i: (i, 0)),
        out_shape=jax.ShapeDtypeStruct((NP, OUT), jnp.float32),
    )(h2, diff, W[:twoD], W[twoD:], b.reshape(1, OUT))


def kernel(node_fts, hidden, gen_dp_data, kill_dp_data, trace_h_i_dp_data,
           cfg_indices_padded, gkt_indices_padded, W, b):
    B, N, D = node_fts.shape
    nf = node_fts[0]
    EG = gkt_indices_padded.shape[1]
    pad = _EG_PAD - EG
    src = jnp.pad(gkt_indices_padded[0, :, 0], (0, pad))
    tgt = jnp.pad(gkt_indices_padded[0, :, 1], (0, pad))
    w2 = jnp.stack([
        jnp.pad(trace_h_i_dp_data[0], (0, pad)),
        jnp.pad(gen_dp_data[0] - kill_dp_data[0], (0, pad)),
    ])

    msg1, diff = _gkt_pass(nf, src, tgt, w2)

    EC = cfg_indices_padded.shape[1]
    cs0 = cfg_indices_padded[0, :, 0]
    ct0 = cfg_indices_padded[0, :, 1]
    bucket = ct0 // _RPW
    order = jnp.argsort(bucket)
    bsort = bucket[order]
    cs = jnp.pad(cs0[order], (0, _EC_PAD + _GCH - EC))
    ct = jnp.pad(ct0[order], (0, _EC_PAD + _GCH - EC), constant_values=_NP)
    starts = jnp.searchsorted(bsort, jnp.arange(33)).astype(jnp.int32)
    starts = jnp.pad(starts, (0, 15), mode="edge")
    h2 = _cfg_pass(msg1, hidden[0], cs, ct, starts)

    out = _fused_projection(h2, diff, W, b)
    return out[None, :N, :]


# speedup vs baseline: 1.0946x; 1.0946x over previous
import functools

import jax
import jax.numpy as jnp
from jax import lax
from jax.experimental import pallas as pl
from jax.experimental.pallas import tpu as pltpu
from jax.experimental.pallas import tpu_sc as plsc

_NC, _NS, _L = 2, 16, 16
_N = 10000
_NP = 10240
_D = 128
_CH = 128
_NCH = 80
_EG_PAD = _NS * _NCH * _CH
_EPT = _EG_PAD // _NS


def _gkt_body(nf, src_h, tgt_h, w_h, msg1_o, diff_o,
              acc, g0, g1, sv0, sv1, tv0, tv1, tv2, tv3, wv0, wv1, wv2, wv3,
              gsem0, gsem1, ssem0, ssem1, asem0, asem1):
    c = lax.axis_index("c")
    s = lax.axis_index("s")
    zero16 = jnp.zeros((_L,), jnp.float32)
    gb = (g0, g1)
    sbuf = (sv0, sv1)
    tbuf = (tv0, tv1, tv2, tv3)
    wbuf = (wv0, wv1, wv2, wv3)
    gsem = (gsem0, gsem1)
    ssem = (ssem0, ssem1)
    asem = (asem0, asem1)

    def memset_row(r, carry):
        for j in range(_D // _L):
            g0[r, pl.ds(j * _L, _L)] = zero16
        return carry

    lax.fori_loop(0, _CH, memset_row, 0)
    rpw = _NP // _NS
    w0 = s * rpw
    for i in range(rpw // _CH):
        pltpu.sync_copy(g0, acc.at[pl.ds(w0 + i * _CH, _CH)])
    plsc.subcore_barrier()

    base = s * _EPT

    def issue_smalls(k, b, q):
        off = base + k * _CH
        pltpu.async_copy(src_h.at[pl.ds(off, _CH)], sbuf[b], asem[b])
        pltpu.async_copy(tgt_h.at[pl.ds(off, _CH)], tbuf[q], asem[b])
        pltpu.async_copy(w_h.at[c, pl.ds(off, _CH)], wbuf[q], asem[b])

    def wait_smalls(k, b, q):
        off = base + k * _CH
        pltpu.make_async_copy(src_h.at[pl.ds(off, _CH)], sbuf[b], asem[b]).wait()
        pltpu.make_async_copy(tgt_h.at[pl.ds(off, _CH)], tbuf[q], asem[b]).wait()
        pltpu.make_async_copy(w_h.at[c, pl.ds(off, _CH)], wbuf[q], asem[b]).wait()

    def issue_gather(b):
        pltpu.async_copy(nf.at[sbuf[b]], gb[b], gsem[b])

    def wait_gather(b):
        pltpu.make_async_copy(nf.at[sbuf[b]], gb[b], gsem[b]).wait()

    def issue_scat(b, q):
        pltpu.async_copy(gb[b], acc.at[tbuf[q]], ssem[b], add=True)

    def wait_scat(b, q):
        pltpu.make_async_copy(gb[b], acc.at[tbuf[q]], ssem[b]).wait()

    def scale(b, q):
        gbuf = gb[b]
        wv = wbuf[q]

        def grp(g, carry):
            wvec = wv[pl.ds(g * _L, _L)]
            for rl in range(_L):
                r = g * _L + rl
                sp = jnp.full((_L,), wvec[rl], dtype=jnp.float32)
                for j in range(_D // _L):
                    sl = pl.ds(j * _L, _L)
                    gbuf[r, sl] = gbuf[r, sl] * sp
            return carry

        lax.fori_loop(0, _CH // _L, grp, 0)

    def step(k, q, first=False, guard=True):
        b = q % 2
        wait_gather(b)

        if guard:
            @pl.when(k + 2 < _NCH)
            def _():
                issue_smalls(k + 2, b, (q + 2) % 4)
        else:
            issue_smalls(k + 2, b, (q + 2) % 4)

        if not first:
            wait_scat(1 - b, (q + 3) % 4)

        if guard:
            @pl.when(k + 1 < _NCH)
            def _():
                wait_smalls(k + 1, 1 - b, (q + 1) % 4)
                issue_gather(1 - b)
        else:
            wait_smalls(k + 1, 1 - b, (q + 1) % 4)
            issue_gather(1 - b)

        scale(b, q)
        issue_scat(b, q)

    issue_smalls(0, 0, 0)
    issue_smalls(1, 1, 1)
    wait_smalls(0, 0, 0)
    issue_gather(0)

    step(0, 0, first=True, guard=False)
    step(1, 1, guard=False)
    step(2, 2, guard=False)
    step(3, 3, guard=False)

    def pipe(j, carry):
        k = j * 4
        step(k, 0)
        step(k + 1, 1)
        step(k + 2, 2)
        step(k + 3, 3)
        return carry

    lax.fori_loop(1, _NCH // 4, pipe, 0)
    wait_scat(1, 3)
    plsc.subcore_barrier()

    @pl.when(c == 0)
    def _():
        pltpu.sync_copy(acc.at[pl.ds(w0, rpw)], msg1_o.at[pl.ds(w0, rpw)])

    @pl.when(c == 1)
    def _():
        pltpu.sync_copy(acc.at[pl.ds(w0, rpw)], diff_o.at[pl.ds(w0, rpw)])


@jax.jit
def _gkt_pass(nf, src, tgt, w2):
    run = pl.kernel(
        _gkt_body,
        out_type=[
            jax.ShapeDtypeStruct((_NP, _D), jnp.float32),
            jax.ShapeDtypeStruct((_NP, _D), jnp.float32),
        ],
        mesh=plsc.VectorSubcoreMesh(core_axis_name="c", subcore_axis_name="s"),
        scratch_types=[
            pltpu.VMEM_SHARED((_NP, _D), jnp.float32),
            pltpu.VMEM((_CH, _D), jnp.float32),
            pltpu.VMEM((_CH, _D), jnp.float32),
            pltpu.VMEM((_CH,), jnp.int32),
            pltpu.VMEM((_CH,), jnp.int32),
            pltpu.VMEM((_CH,), jnp.int32),
            pltpu.VMEM((_CH,), jnp.int32),
            pltpu.VMEM((_CH,), jnp.int32),
            pltpu.VMEM((_CH,), jnp.int32),
            pltpu.VMEM((_CH,), jnp.float32),
            pltpu.VMEM((_CH,), jnp.float32),
            pltpu.VMEM((_CH,), jnp.float32),
            pltpu.VMEM((_CH,), jnp.float32),
            pltpu.SemaphoreType.DMA,
            pltpu.SemaphoreType.DMA,
            pltpu.SemaphoreType.DMA,
            pltpu.SemaphoreType.DMA,
            pltpu.SemaphoreType.DMA,
            pltpu.SemaphoreType.DMA,
        ],
    )
    return run(nf, src, tgt, w2)


_EC_PAD = 20480
_RPW = _NP // (_NC * _NS)
_ACC_R = _RPW + 8
_GCH = 64


def _cfg_body(msg1_h, hid_h, cs_h, ct_h, st_h, h2_o,
              acc, stv, eidx, etgt, g1, g2, sem):
    c = lax.axis_index("c")
    s = lax.axis_index("s")
    wid = s * _NC + c
    lo = wid * _RPW
    ninf = jnp.full((_L,), -jnp.inf, dtype=jnp.float32)
    zerof = jnp.zeros((_L,), dtype=jnp.float32)
    lov = jnp.full((_L,), lo, dtype=jnp.int32)
    onev = jnp.full((_L,), 1, dtype=jnp.int32)
    zerov = jnp.full((_L,), 0, dtype=jnp.int32)
    rpwv = jnp.full((_L,), _RPW, dtype=jnp.int32)

    def init_row(r, carry):
        for j in range(2 * _D // _L):
            acc[r, pl.ds(j * _L, _L)] = ninf
        return carry

    lax.fori_loop(0, _ACC_R, init_row, 0)

    pltpu.sync_copy(st_h, stv)
    sv_ = stv[pl.ds(wid, _L)]
    s_lo = sv_[0]
    s_hi = sv_[1]
    a_lo = (s_lo // 8) * 8
    ng = (s_hi - a_lo + _GCH - 1) // _GCH

    def grp(g, carry):
        off = a_lo + g * _GCH
        pltpu.async_copy(cs_h.at[pl.ds(off, _GCH)], eidx, sem)
        pltpu.async_copy(ct_h.at[pl.ds(off, _GCH)], etgt, sem)
        pltpu.make_async_copy(cs_h.at[pl.ds(off, _GCH)], eidx, sem).wait()
        pltpu.make_async_copy(ct_h.at[pl.ds(off, _GCH)], etgt, sem).wait()
        pltpu.async_copy(msg1_h.at[eidx], g1, sem)
        pltpu.async_copy(hid_h.at[eidx], g2, sem)
        pltpu.make_async_copy(msg1_h.at[eidx], g1, sem).wait()
        pltpu.make_async_copy(hid_h.at[eidx], g2, sem).wait()
        for vq in range(_GCH // _L):
            tv = etgt[pl.ds(vq * _L, _L)]
            d = tv - lov
            ge0 = jnp.minimum(jnp.maximum(d + onev, zerov), onev)
            ltr = jnp.minimum(jnp.maximum(rpwv - d, zerov), onev)
            mi = ge0 * ltr
            tlv = mi * d + (onev - mi) * rpwv
            for rl in range(_L):
                r = vq * _L + rl
                tl = tlv[rl]
                for j in range(_D // _L):
                    sl = pl.ds(j * _L, _L)
                    acc[tl, sl] = jnp.maximum(acc[tl, sl], g1[r, sl])
                for j in range(_D // _L):
                    sl = pl.ds(j * _L, _L)
                    sl2 = pl.ds(_D + j * _L, _L)
                    acc[tl, sl2] = jnp.maximum(acc[tl, sl2], g2[r, sl])
        return carry

    lax.fori_loop(0, ng, grp, 0)

    def fin_row(r, carry):
        for j in range(2 * _D // _L):
            sl = pl.ds(j * _L, _L)
            v = acc[r, sl]
            acc[r, sl] = jnp.where(v > ninf, v, zerof)
        return carry

    lax.fori_loop(0, _RPW, fin_row, 0)
    pltpu.sync_copy(acc.at[pl.ds(0, _RPW)], h2_o.at[pl.ds(lo, _RPW)])


@jax.jit
def _cfg_pass(msg1, hid, cs, ct, starts):
    run = pl.kernel(
        _cfg_body,
        out_type=jax.ShapeDtypeStruct((_NP, 2 * _D), jnp.float32),
        mesh=plsc.VectorSubcoreMesh(core_axis_name="c", subcore_axis_name="s"),
        scratch_types=[
            pltpu.VMEM((_ACC_R, 2 * _D), jnp.float32),
            pltpu.VMEM((48,), jnp.int32),
            pltpu.VMEM((_GCH,), jnp.int32),
            pltpu.VMEM((_GCH,), jnp.int32),
            pltpu.VMEM((_GCH, _D), jnp.float32),
            pltpu.VMEM((_GCH, _D), jnp.float32),
            pltpu.SemaphoreType.DMA,
        ],
    )
    return run(msg1, hid, cs, ct, starts)


def _proj_body(h2_ref, diff_ref, w2_ref, wd_ref, b_ref, o_ref):
    acc = jnp.dot(h2_ref[...], w2_ref[...], preferred_element_type=jnp.float32)
    acc = acc + jnp.dot(diff_ref[...], wd_ref[...], preferred_element_type=jnp.float32)
    o_ref[...] = jnp.maximum(acc + b_ref[...], 0.0)


def _fused_projection(h2, diff, W, b):
    NP, twoD = h2.shape
    D = diff.shape[1]
    OUT = W.shape[1]
    BN = 512
    return pl.pallas_call(
        _proj_body,
        grid=(NP // BN,),
        in_specs=[
            pl.BlockSpec((BN, twoD), lambda i: (i, 0)),
            pl.BlockSpec((BN, D), lambda i: (i, 0)),
            pl.BlockSpec((twoD, OUT), lambda i: (0, 0)),
            pl.BlockSpec((D, OUT), lambda i: (0, 0)),
            pl.BlockSpec((1, OUT), lambda i: (0, 0)),
        ],
        out_specs=pl.BlockSpec((BN, OUT), lambda i: (i, 0)),
        out_shape=jax.ShapeDtypeStruct((NP, OUT), jnp.float32),
    )(h2, diff, W[:twoD], W[twoD:], b.reshape(1, OUT))


def kernel(node_fts, hidden, gen_dp_data, kill_dp_data, trace_h_i_dp_data,
           cfg_indices_padded, gkt_indices_padded, W, b):
    B, N, D = node_fts.shape
    nf = node_fts[0]
    EG = gkt_indices_padded.shape[1]
    pad = _EG_PAD - EG
    src = jnp.pad(gkt_indices_padded[0, :, 0], (0, pad))
    tgt = jnp.pad(gkt_indices_padded[0, :, 1], (0, pad))
    w2 = jnp.stack([
        jnp.pad(trace_h_i_dp_data[0], (0, pad)),
        jnp.pad(gen_dp_data[0] - kill_dp_data[0], (0, pad)),
    ])

    msg1, diff = _gkt_pass(nf, src, tgt, w2)

    EC = cfg_indices_padded.shape[1]
    cs0 = cfg_indices_padded[0, :, 0]
    ct0 = cfg_indices_padded[0, :, 1]
    bucket = ct0 // _RPW
    order = jnp.argsort(bucket)
    bsort = bucket[order]
    cs = jnp.pad(cs0[order], (0, _EC_PAD + _GCH - EC))
    ct = jnp.pad(ct0[order], (0, _EC_PAD + _GCH - EC), constant_values=_NP)
    starts = jnp.searchsorted(bsort, jnp.arange(33)).astype(jnp.int32)
    starts = jnp.pad(starts, (0, 15), mode="edge")
    h2 = _cfg_pass(msg1, hidden[0], cs, ct, starts)

    out = _fused_projection(h2, diff, W, b)
    return out[None, :N, :]


# speedup vs baseline: 1.0947x; 1.0001x over previous
import jax
import jax.numpy as jnp
from jax import lax
from jax.experimental import pallas as pl
from jax.experimental.pallas import tpu as pltpu
from jax.experimental.pallas import tpu_sc as plsc

_NC, _NS, _L = 2, 16, 16
_N = 10000
_NP = 10240
_D = 128
_CH = 128
_NCH = 80
_EG_PAD = _NS * _NCH * _CH
_EPT = _EG_PAD // _NS


def _gkt_body(nf, src_h, tgt_h, w_h, msg1_o, diff_o,
              acc, g0, g1, sv0, sv1, tv0, tv1, tv2, tv3, wv0, wv1, wv2, wv3,
              gsem0, gsem1, ssem0, ssem1, asem0, asem1):
    c = lax.axis_index("c")
    s = lax.axis_index("s")
    zero16 = jnp.zeros((_L,), jnp.float32)
    gb = (g0, g1)
    sbuf = (sv0, sv1)
    tbuf = (tv0, tv1, tv2, tv3)
    wbuf = (wv0, wv1, wv2, wv3)
    gsem = (gsem0, gsem1)
    ssem = (ssem0, ssem1)
    asem = (asem0, asem1)

    def memset_row(r, carry):
        for j in range(_D // _L):
            g0[r, pl.ds(j * _L, _L)] = zero16
        return carry

    lax.fori_loop(0, _CH, memset_row, 0)
    rpw = _NP // _NS
    w0 = s * rpw
    for i in range(rpw // _CH):
        pltpu.sync_copy(g0, acc.at[pl.ds(w0 + i * _CH, _CH)])
    plsc.subcore_barrier()

    base = s * _EPT

    def issue_smalls(k, b, q):
        off = base + k * _CH
        pltpu.async_copy(src_h.at[pl.ds(off, _CH)], sbuf[b], asem[b])
        pltpu.async_copy(tgt_h.at[pl.ds(off, _CH)], tbuf[q], asem[b])
        pltpu.async_copy(w_h.at[c, pl.ds(off, _CH)], wbuf[q], asem[b])

    def wait_smalls(k, b, q):
        off = base + k * _CH
        pltpu.make_async_copy(src_h.at[pl.ds(off, _CH)], sbuf[b], asem[b]).wait()
        pltpu.make_async_copy(tgt_h.at[pl.ds(off, _CH)], tbuf[q], asem[b]).wait()
        pltpu.make_async_copy(w_h.at[c, pl.ds(off, _CH)], wbuf[q], asem[b]).wait()

    def issue_gather(b):
        pltpu.async_copy(nf.at[sbuf[b]], gb[b], gsem[b])

    def wait_gather(b):
        pltpu.make_async_copy(nf.at[sbuf[b]], gb[b], gsem[b]).wait()

    def issue_scat(b, q):
        pltpu.async_copy(gb[b], acc.at[tbuf[q]], ssem[b], add=True)

    def wait_scat(b, q):
        pltpu.make_async_copy(gb[b], acc.at[tbuf[q]], ssem[b]).wait()

    def scale(b, q):
        gbuf = gb[b]
        wv = wbuf[q]

        def grp(g, carry):
            wvec = wv[pl.ds(g * _L, _L)]
            for rl in range(_L):
                r = g * _L + rl
                sp = jnp.full((_L,), wvec[rl], dtype=jnp.float32)
                for j in range(_D // _L):
                    sl = pl.ds(j * _L, _L)
                    gbuf[r, sl] = gbuf[r, sl] * sp
            return carry

        lax.fori_loop(0, _CH // _L, grp, 0)

    def step(k, q, first=False, guard=True):
        b = q % 2
        wait_gather(b)

        if guard:
            @pl.when(k + 2 < _NCH)
            def _():
                issue_smalls(k + 2, b, (q + 2) % 4)
        else:
            issue_smalls(k + 2, b, (q + 2) % 4)

        if not first:
            wait_scat(1 - b, (q + 3) % 4)

        if guard:
            @pl.when(k + 1 < _NCH)
            def _():
                wait_smalls(k + 1, 1 - b, (q + 1) % 4)
                issue_gather(1 - b)
        else:
            wait_smalls(k + 1, 1 - b, (q + 1) % 4)
            issue_gather(1 - b)

        scale(b, q)
        issue_scat(b, q)

    issue_smalls(0, 0, 0)
    issue_smalls(1, 1, 1)
    wait_smalls(0, 0, 0)
    issue_gather(0)

    step(0, 0, first=True, guard=False)
    step(1, 1, guard=False)
    step(2, 2, guard=False)
    step(3, 3, guard=False)

    def pipe(j, carry):
        k = j * 4
        step(k, 0)
        step(k + 1, 1)
        step(k + 2, 2)
        step(k + 3, 3)
        return carry

    lax.fori_loop(1, _NCH // 4, pipe, 0)
    wait_scat(1, 3)
    plsc.subcore_barrier()

    @pl.when(c == 0)
    def _():
        pltpu.sync_copy(acc.at[pl.ds(w0, rpw)], msg1_o.at[pl.ds(w0, rpw)])

    @pl.when(c == 1)
    def _():
        pltpu.sync_copy(acc.at[pl.ds(w0, rpw)], diff_o.at[pl.ds(w0, rpw)])


@jax.jit
def _gkt_pass(nf, src, tgt, w2):
    run = pl.kernel(
        _gkt_body,
        out_type=[
            jax.ShapeDtypeStruct((_NP, _D), jnp.float32),
            jax.ShapeDtypeStruct((_NP, _D), jnp.float32),
        ],
        mesh=plsc.VectorSubcoreMesh(core_axis_name="c", subcore_axis_name="s"),
        scratch_types=[
            pltpu.VMEM_SHARED((_NP, _D), jnp.float32),
            pltpu.VMEM((_CH, _D), jnp.float32),
            pltpu.VMEM((_CH, _D), jnp.float32),
            pltpu.VMEM((_CH,), jnp.int32),
            pltpu.VMEM((_CH,), jnp.int32),
            pltpu.VMEM((_CH,), jnp.int32),
            pltpu.VMEM((_CH,), jnp.int32),
            pltpu.VMEM((_CH,), jnp.int32),
            pltpu.VMEM((_CH,), jnp.int32),
            pltpu.VMEM((_CH,), jnp.float32),
            pltpu.VMEM((_CH,), jnp.float32),
            pltpu.VMEM((_CH,), jnp.float32),
            pltpu.VMEM((_CH,), jnp.float32),
            pltpu.SemaphoreType.DMA,
            pltpu.SemaphoreType.DMA,
            pltpu.SemaphoreType.DMA,
            pltpu.SemaphoreType.DMA,
            pltpu.SemaphoreType.DMA,
            pltpu.SemaphoreType.DMA,
        ],
    )
    return run(nf, src, tgt, w2)


_EC_PAD = 20480
_RPW = _NP // (_NC * _NS)
_ACC_R = _RPW + 8
_GCH = 64


def _cfg_body(msg1_h, hid_h, cs_h, ct_h, st_h, h2_o,
              acc, stv, eidx, etgt, g1, g2, sem):
    c = lax.axis_index("c")
    s = lax.axis_index("s")
    wid = s * _NC + c
    lo = wid * _RPW
    ninf = jnp.full((_L,), -jnp.inf, dtype=jnp.float32)
    zerof = jnp.zeros((_L,), dtype=jnp.float32)
    lov = jnp.full((_L,), lo, dtype=jnp.int32)
    onev = jnp.full((_L,), 1, dtype=jnp.int32)
    zerov = jnp.full((_L,), 0, dtype=jnp.int32)
    rpwv = jnp.full((_L,), _RPW, dtype=jnp.int32)

    def init_row(r, carry):
        for j in range(2 * _D // _L):
            acc[r, pl.ds(j * _L, _L)] = ninf
        return carry

    lax.fori_loop(0, _ACC_R, init_row, 0)

    pltpu.sync_copy(st_h, stv)
    sv_ = stv[pl.ds(wid, _L)]
    s_lo = sv_[0]
    s_hi = sv_[1]
    a_lo = (s_lo // 8) * 8
    ng = (s_hi - a_lo + _GCH - 1) // _GCH

    def grp(g, carry):
        off = a_lo + g * _GCH
        pltpu.async_copy(cs_h.at[pl.ds(off, _GCH)], eidx, sem)
        pltpu.async_copy(ct_h.at[pl.ds(off, _GCH)], etgt, sem)
        pltpu.make_async_copy(cs_h.at[pl.ds(off, _GCH)], eidx, sem).wait()
        pltpu.make_async_copy(ct_h.at[pl.ds(off, _GCH)], etgt, sem).wait()
        pltpu.async_copy(msg1_h.at[eidx], g1, sem)
        pltpu.async_copy(hid_h.at[eidx], g2, sem)
        pltpu.make_async_copy(msg1_h.at[eidx], g1, sem).wait()
        pltpu.make_async_copy(hid_h.at[eidx], g2, sem).wait()
        for vq in range(_GCH // _L):
            tv = etgt[pl.ds(vq * _L, _L)]
            d = tv - lov
            ge0 = jnp.minimum(jnp.maximum(d + onev, zerov), onev)
            ltr = jnp.minimum(jnp.maximum(rpwv - d, zerov), onev)
            mi = ge0 * ltr
            tlv = mi * d + (onev - mi) * rpwv
            for rl in range(_L):
                r = vq * _L + rl
                tl = tlv[rl]
                for j in range(_D // _L):
                    sl = pl.ds(j * _L, _L)
                    acc[tl, sl] = jnp.maximum(acc[tl, sl], g1[r, sl])
                for j in range(_D // _L):
                    sl = pl.ds(j * _L, _L)
                    sl2 = pl.ds(_D + j * _L, _L)
                    acc[tl, sl2] = jnp.maximum(acc[tl, sl2], g2[r, sl])
        return carry

    lax.fori_loop(0, ng, grp, 0)

    def fin_row(r, carry):
        for j in range(2 * _D // _L):
            sl = pl.ds(j * _L, _L)
            v = acc[r, sl]
            acc[r, sl] = jnp.where(v > ninf, v, zerof)
        return carry

    lax.fori_loop(0, _RPW, fin_row, 0)
    pltpu.sync_copy(acc.at[pl.ds(0, _RPW)], h2_o.at[pl.ds(lo, _RPW)])


@jax.jit
def _cfg_pass(msg1, hid, cs, ct, starts):
    run = pl.kernel(
        _cfg_body,
        out_type=jax.ShapeDtypeStruct((_NP, 2 * _D), jnp.float32),
        mesh=plsc.VectorSubcoreMesh(core_axis_name="c", subcore_axis_name="s"),
        scratch_types=[
            pltpu.VMEM((_ACC_R, 2 * _D), jnp.float32),
            pltpu.VMEM((48,), jnp.int32),
            pltpu.VMEM((_GCH,), jnp.int32),
            pltpu.VMEM((_GCH,), jnp.int32),
            pltpu.VMEM((_GCH, _D), jnp.float32),
            pltpu.VMEM((_GCH, _D), jnp.float32),
            pltpu.SemaphoreType.DMA,
        ],
    )
    return run(msg1, hid, cs, ct, starts)


def _proj_body(h2_ref, diff_ref, w2_ref, wd_ref, b_ref, o_ref):
    acc = jnp.dot(h2_ref[...], w2_ref[...], preferred_element_type=jnp.float32)
    acc = acc + jnp.dot(diff_ref[...], wd_ref[...], preferred_element_type=jnp.float32)
    o_ref[...] = jnp.maximum(acc + b_ref[...], 0.0)


def _fused_projection(h2, diff, W, b):
    NP, twoD = h2.shape
    D = diff.shape[1]
    OUT = W.shape[1]
    BN = 512
    return pl.pallas_call(
        _proj_body,
        grid=(NP // BN,),
        in_specs=[
            pl.BlockSpec((BN, twoD), lambda i: (i, 0)),
            pl.BlockSpec((BN, D), lambda i: (i, 0)),
            pl.BlockSpec((twoD, OUT), lambda i: (0, 0)),
            pl.BlockSpec((D, OUT), lambda i: (0, 0)),
            pl.BlockSpec((1, OUT), lambda i: (0, 0)),
        ],
        out_specs=pl.BlockSpec((BN, OUT), lambda i: (i, 0)),
        out_shape=jax.ShapeDtypeStruct((NP, OUT), jnp.float32),
    )(h2, diff, W[:twoD], W[twoD:], b.reshape(1, OUT))


def kernel(node_fts, hidden, gen_dp_data, kill_dp_data, trace_h_i_dp_data,
           cfg_indices_padded, gkt_indices_padded, W, b):
    B, N, D = node_fts.shape
    nf = node_fts[0]
    EG = gkt_indices_padded.shape[1]
    pad = _EG_PAD - EG
    src = jnp.pad(gkt_indices_padded[0, :, 0], (0, pad))
    tgt = jnp.pad(gkt_indices_padded[0, :, 1], (0, pad))
    w2 = jnp.stack([
        jnp.pad(trace_h_i_dp_data[0], (0, pad)),
        jnp.pad(gen_dp_data[0] - kill_dp_data[0], (0, pad)),
    ])

    msg1, diff = _gkt_pass(nf, src, tgt, w2)

    EC = cfg_indices_padded.shape[1]
    cs0 = cfg_indices_padded[0, :, 0]
    ct0 = cfg_indices_padded[0, :, 1]
    bucket = ct0 // _RPW
    order = jnp.argsort(bucket)
    bsort = bucket[order]
    cs = jnp.pad(cs0[order], (0, _EC_PAD + _GCH - EC))
    ct = jnp.pad(ct0[order], (0, _EC_PAD + _GCH - EC), constant_values=_NP)
    starts = jnp.searchsorted(bsort, jnp.arange(33)).astype(jnp.int32)
    starts = jnp.pad(starts, (0, 15), mode="edge")
    h2 = _cfg_pass(msg1, hidden[0], cs, ct, starts)

    out = _fused_projection(h2, diff, W, b)
    return out[None, :N, :]
